# baseline (speedup 1.0000x reference)
import functools

import jax
import jax.numpy as jnp
from jax import lax
from jax.experimental import pallas as pl
from jax.experimental.pallas import tpu as pltpu

_LN_EPS = 1e-5
_BS = 4


_GELU_K1 = -2.0 * 0.7978845608028654 * 1.4426950408889634
_GELU_K3 = _GELU_K1 * 0.044715


def _gelu(x):
    x2 = x * x
    u = x * (_GELU_K1 + _GELU_K3 * x2)
    return x / (1.0 + jnp.exp2(u))


def _ln_stats(x, jones):
    zb = x.astype(jnp.bfloat16)
    m = jnp.dot(zb, jones, preferred_element_type=jnp.float32)
    msq = jnp.dot(zb * zb, jones, preferred_element_type=jnp.float32)
    var = msq - m * m
    return m, lax.rsqrt(var + _LN_EPS)


def _mixer_body(depth, bs,
                x_ref, jones_ref, ln1g, ln1b, wt1, bt1, wt2, bt2,
                wc1p, bc1p, wc2t, bc2, whp, bhp, o_ref):
    bf16 = jnp.bfloat16
    f32 = jnp.float32
    jones = jones_ref[...]
    for b in range(bs):
        x = x_ref[b]
        for d in range(depth):
            m, r = _ln_stats(x, jones)
            y = ((x - m) * r * ln1g[d] + ln1b[d]).astype(bf16)
            h = jnp.dot(wt1[d], y, preferred_element_type=f32) + bt1[d]
            h = _gelu(h).astype(bf16)
            x = x + (jnp.dot(wt2[d], h, preferred_element_type=f32)
                     + bt2[d])
            m, r = _ln_stats(x, jones)
            z = ((x - m) * r).astype(bf16)
            h = jnp.dot(z, wc1p[d], preferred_element_type=f32) + bc1p[d]
            h = _gelu(h).astype(bf16)
            x = x + (jnp.dot(h, wc2t[d], preferred_element_type=f32)
                     + bc2[d])
        m, r = _ln_stats(x, jones)
        z = ((x - m) * r).astype(bf16)
        o_ref[b] = jnp.dot(z, whp[...], preferred_element_type=f32) + bhp[...]


def _rep_spec(shape):
    nd = len(shape)
    return pl.BlockSpec(shape, lambda i, _n=nd: (0,) * _n)


@jax.jit
def kernel(x, ln1_g, ln1_b, wt1, bt1, wt2, bt2, ln2_g, ln2_b,
           wc1, bc1, wc2, bc2, lnf_g, lnf_b, wh, bh):
    b, n_patch, dim = x.shape
    depth = wt1.shape[0]
    n_out = wh.shape[0]
    f32, bf16 = jnp.float32, jnp.bfloat16
    bs = _BS if b % _BS == 0 else 1

    wc1t = jnp.transpose(wc1.astype(f32), (0, 2, 1))
    wc1p = (ln2_g.astype(f32)[:, :, None] * wc1t).astype(bf16)
    bc1p = (bc1.astype(f32)
            + jnp.einsum('dk,dkc->dc', ln2_b.astype(f32), wc1t))[:, None, :]
    wc2t = jnp.transpose(wc2.astype(f32), (0, 2, 1)).astype(bf16)
    wht = wh.astype(f32).T
    whp = (lnf_g.astype(f32)[:, None] * wht).astype(bf16)
    bhp = (bh.astype(f32) + lnf_b.astype(f32) @ wht)[None, :]

    prepped = [
        jnp.full((dim, dim), 1.0 / dim, dtype=bf16),
        ln1_g.astype(f32).reshape(depth, 1, dim),
        ln1_b.astype(f32).reshape(depth, 1, dim),
        wt1.astype(bf16),
        bt1.astype(f32)[:, :, None],
        wt2.astype(bf16),
        bt2.astype(f32)[:, :, None],
        wc1p, bc1p,
        wc2t,
        bc2.astype(f32)[:, None, :],
        whp, bhp,
    ]

    in_specs = [pl.BlockSpec((bs, n_patch, dim), lambda i: (i, 0, 0))]
    in_specs += [_rep_spec(a.shape) for a in prepped]

    return pl.pallas_call(
        functools.partial(_mixer_body, depth, bs),
        out_shape=jax.ShapeDtypeStruct((b, n_patch, n_out), f32),
        grid=(b // bs,),
        in_specs=in_specs,
        out_specs=pl.BlockSpec((bs, n_patch, n_out), lambda i: (i, 0, 0)),
        compiler_params=pltpu.CompilerParams(
            dimension_semantics=("parallel",)),
    )(x.astype(f32), *prepped)


# speedup vs baseline: 1.1440x; 1.1440x over previous
import functools

import jax
import jax.numpy as jnp
from jax import lax
from jax.experimental import pallas as pl
from jax.experimental.pallas import tpu as pltpu

_LN_EPS = 1e-5
_BS = 4


_GELU_KS = -1.702 * 1.4426950408889634


def _gelu(x):
    return x / (1.0 + jnp.exp2(_GELU_KS * x))


def _ln_stats(x):
    m = jnp.mean(x, axis=-1, keepdims=True)
    xc = x - m
    var = jnp.mean(xc * xc, axis=-1, keepdims=True)
    return m, lax.rsqrt(var + _LN_EPS)


def _mixer_body(depth, bs,
                x_ref, ln1g, ln1b, wt1, bt1, wt2, bt2,
                wc1p, bc1p, wc2t, bc2, whp, bhp, o_ref):
    bf16 = jnp.bfloat16
    f32 = jnp.float32
    for b in range(bs):
        x = x_ref[b]
        for d in range(depth):
            m, r = _ln_stats(x)
            y = ((x - m) * r * ln1g[d] + ln1b[d]).astype(bf16)
            h = jnp.dot(wt1[d], y, preferred_element_type=f32) + bt1[d]
            h = _gelu(h).astype(bf16)
            x = x + (jnp.dot(wt2[d], h, preferred_element_type=f32)
                     + bt2[d])
            m, r = _ln_stats(x)
            z = ((x - m) * r).astype(bf16)
            h = jnp.dot(z, wc1p[d], preferred_element_type=f32) + bc1p[d]
            h = _gelu(h).astype(bf16)
            x = x + (jnp.dot(h, wc2t[d], preferred_element_type=f32)
                     + bc2[d])
        m, r = _ln_stats(x)
        z = ((x - m) * r).astype(bf16)
        o_ref[b] = jnp.dot(z, whp[...], preferred_element_type=f32) + bhp[...]


def _rep_spec(shape):
    nd = len(shape)
    return pl.BlockSpec(shape, lambda i, _n=nd: (0,) * _n)


@jax.jit
def kernel(x, ln1_g, ln1_b, wt1, bt1, wt2, bt2, ln2_g, ln2_b,
           wc1, bc1, wc2, bc2, lnf_g, lnf_b, wh, bh):
    b, n_patch, dim = x.shape
    depth = wt1.shape[0]
    n_out = wh.shape[0]
    f32, bf16 = jnp.float32, jnp.bfloat16
    bs = _BS if b % _BS == 0 else 1

    wc1t = jnp.transpose(wc1.astype(f32), (0, 2, 1))
    wc1p = (ln2_g.astype(f32)[:, :, None] * wc1t).astype(bf16)
    bc1p = (bc1.astype(f32)
            + jnp.einsum('dk,dkc->dc', ln2_b.astype(f32), wc1t))[:, None, :]
    wc2t = jnp.transpose(wc2.astype(f32), (0, 2, 1)).astype(bf16)
    wht = wh.astype(f32).T
    whp = (lnf_g.astype(f32)[:, None] * wht).astype(bf16)
    bhp = (bh.astype(f32) + lnf_b.astype(f32) @ wht)[None, :]

    prepped = [
        ln1_g.astype(f32).reshape(depth, 1, dim),
        ln1_b.astype(f32).reshape(depth, 1, dim),
        wt1.astype(bf16),
        bt1.astype(f32)[:, :, None],
        wt2.astype(bf16),
        bt2.astype(f32)[:, :, None],
        wc1p, bc1p,
        wc2t,
        bc2.astype(f32)[:, None, :],
        whp, bhp,
    ]

    in_specs = [pl.BlockSpec((bs, n_patch, dim), lambda i: (i, 0, 0))]
    in_specs += [_rep_spec(a.shape) for a in prepped]

    return pl.pallas_call(
        functools.partial(_mixer_body, depth, bs),
        out_shape=jax.ShapeDtypeStruct((b, n_patch, n_out), f32),
        grid=(b // bs,),
        in_specs=in_specs,
        out_specs=pl.BlockSpec((bs, n_patch, n_out), lambda i: (i, 0, 0)),
        compiler_params=pltpu.CompilerParams(
            dimension_semantics=("parallel",)),
    )(x.astype(f32), *prepped)


# speedup vs baseline: 1.1729x; 1.0252x over previous
import functools

import jax
import jax.numpy as jnp
from jax import lax
from jax.experimental import pallas as pl
from jax.experimental.pallas import tpu as pltpu

_LN_EPS = 1e-5
_BS = 4


def _gelu(x):
    return 0.5 * x * (1.0 + lax.erf(x * 0.7071067811865476))


def _ln_stats(x):
    m = jnp.mean(x, axis=-1, keepdims=True)
    xc = x - m
    var = jnp.mean(xc * xc, axis=-1, keepdims=True)
    return m, lax.rsqrt(var + _LN_EPS)


def _mixer_body(depth, bs,
                x_ref, ln1g, ln1b, wt1, bt1, wt2, bt2,
                wc1p, bc1p, wc2t, bc2, whp, bhp, o_ref):
    bf16 = jnp.bfloat16
    f32 = jnp.float32
    for b in range(bs):
        x = x_ref[b]
        for d in range(depth):
            m, r = _ln_stats(x)
            y = ((x - m) * r * ln1g[d] + ln1b[d]).astype(bf16)
            h = jnp.dot(wt1[d], y, preferred_element_type=f32) + bt1[d]
            h = _gelu(h).astype(bf16)
            x = x + (jnp.dot(wt2[d], h, preferred_element_type=f32)
                     + bt2[d])
            m, r = _ln_stats(x)
            z = ((x - m) * r).astype(bf16)
            h = jnp.dot(z, wc1p[d], preferred_element_type=f32) + bc1p[d]
            h = _gelu(h).astype(bf16)
            x = x + (jnp.dot(h, wc2t[d], preferred_element_type=f32)
                     + bc2[d])
        m, r = _ln_stats(x)
        z = ((x - m) * r).astype(bf16)
        o_ref[b] = jnp.dot(z, whp[...], preferred_element_type=f32) + bhp[...]


def _rep_spec(shape):
    nd = len(shape)
    return pl.BlockSpec(shape, lambda i, _n=nd: (0,) * _n)


@jax.jit
def kernel(x, ln1_g, ln1_b, wt1, bt1, wt2, bt2, ln2_g, ln2_b,
           wc1, bc1, wc2, bc2, lnf_g, lnf_b, wh, bh):
    b, n_patch, dim = x.shape
    depth = wt1.shape[0]
    n_out = wh.shape[0]
    f32, bf16 = jnp.float32, jnp.bfloat16
    bs = _BS if b % _BS == 0 else 1

    wc1t = jnp.transpose(wc1.astype(f32), (0, 2, 1))
    wc1p = (ln2_g.astype(f32)[:, :, None] * wc1t).astype(bf16)
    bc1p = (bc1.astype(f32)
            + jnp.einsum('dk,dkc->dc', ln2_b.astype(f32), wc1t))[:, None, :]
    wc2t = jnp.transpose(wc2.astype(f32), (0, 2, 1)).astype(bf16)
    wht = wh.astype(f32).T
    whp = (lnf_g.astype(f32)[:, None] * wht).astype(bf16)
    bhp = (bh.astype(f32) + lnf_b.astype(f32) @ wht)[None, :]

    prepped = [
        ln1_g.astype(f32).reshape(depth, 1, dim),
        ln1_b.astype(f32).reshape(depth, 1, dim),
        wt1.astype(bf16),
        bt1.astype(f32)[:, :, None],
        wt2.astype(bf16),
        bt2.astype(f32)[:, :, None],
        wc1p, bc1p,
        wc2t,
        bc2.astype(f32)[:, None, :],
        whp, bhp,
    ]

    in_specs = [pl.BlockSpec((bs, n_patch, dim), lambda i: (i, 0, 0))]
    in_specs += [_rep_spec(a.shape) for a in prepped]

    return pl.pallas_call(
        functools.partial(_mixer_body, depth, bs),
        out_shape=jax.ShapeDtypeStruct((b, n_patch, n_out), f32),
        grid=(b // bs,),
        in_specs=in_specs,
        out_specs=pl.BlockSpec((bs, n_patch, n_out), lambda i: (i, 0, 0)),
        compiler_params=pltpu.CompilerParams(
            dimension_semantics=("parallel",)),
    )(x.astype(f32), *prepped)


# speedup vs baseline: 1.1934x; 1.0175x over previous
import functools

import jax
import jax.numpy as jnp
from jax import lax
from jax.experimental import pallas as pl
from jax.experimental.pallas import tpu as pltpu

_LN_EPS = 1e-5
_BS = 8


def _gelu(x):
    return 0.5 * x * (1.0 + lax.erf(x * 0.7071067811865476))


def _ln_stats(x):
    m = jnp.mean(x, axis=-1, keepdims=True)
    xc = x - m
    var = jnp.mean(xc * xc, axis=-1, keepdims=True)
    return m, lax.rsqrt(var + _LN_EPS)


def _mixer_body(depth, bs,
                x_ref, ln1g, ln1b, wt1, bt1, wt2, bt2,
                wc1p, bc1p, wc2t, bc2, whp, bhp, o_ref):
    bf16 = jnp.bfloat16
    f32 = jnp.float32
    for b in range(bs):
        x = x_ref[b]
        for d in range(depth):
            m, r = _ln_stats(x)
            y = ((x - m) * r * ln1g[d] + ln1b[d]).astype(bf16)
            h = jnp.dot(wt1[d], y, preferred_element_type=f32) + bt1[d]
            h = _gelu(h).astype(bf16)
            x = x + (jnp.dot(wt2[d], h, preferred_element_type=f32)
                     + bt2[d])
            m, r = _ln_stats(x)
            z = ((x - m) * r).astype(bf16)
            h = jnp.dot(z, wc1p[d], preferred_element_type=f32) + bc1p[d]
            h = _gelu(h).astype(bf16)
            x = x + (jnp.dot(h, wc2t[d], preferred_element_type=f32)
                     + bc2[d])
        m, r = _ln_stats(x)
        z = ((x - m) * r).astype(bf16)
        o_ref[b] = jnp.dot(z, whp[...], preferred_element_type=f32) + bhp[...]


def _rep_spec(shape):
    nd = len(shape)
    return pl.BlockSpec(shape, lambda i, _n=nd: (0,) * _n)


@jax.jit
def kernel(x, ln1_g, ln1_b, wt1, bt1, wt2, bt2, ln2_g, ln2_b,
           wc1, bc1, wc2, bc2, lnf_g, lnf_b, wh, bh):
    b, n_patch, dim = x.shape
    depth = wt1.shape[0]
    n_out = wh.shape[0]
    f32, bf16 = jnp.float32, jnp.bfloat16
    bs = _BS if b % _BS == 0 else 1

    wc1t = jnp.transpose(wc1.astype(f32), (0, 2, 1))
    wc1p = (ln2_g.astype(f32)[:, :, None] * wc1t).astype(bf16)
    bc1p = (bc1.astype(f32)
            + jnp.einsum('dk,dkc->dc', ln2_b.astype(f32), wc1t))[:, None, :]
    wc2t = jnp.transpose(wc2.astype(f32), (0, 2, 1)).astype(bf16)
    wht = wh.astype(f32).T
    whp = (lnf_g.astype(f32)[:, None] * wht).astype(bf16)
    bhp = (bh.astype(f32) + lnf_b.astype(f32) @ wht)[None, :]

    prepped = [
        ln1_g.astype(f32).reshape(depth, 1, dim),
        ln1_b.astype(f32).reshape(depth, 1, dim),
        wt1.astype(bf16),
        bt1.astype(f32)[:, :, None],
        wt2.astype(bf16),
        bt2.astype(f32)[:, :, None],
        wc1p, bc1p,
        wc2t,
        bc2.astype(f32)[:, None, :],
        whp, bhp,
    ]

    in_specs = [pl.BlockSpec((bs, n_patch, dim), lambda i: (i, 0, 0))]
    in_specs += [_rep_spec(a.shape) for a in prepped]

    return pl.pallas_call(
        functools.partial(_mixer_body, depth, bs),
        out_shape=jax.ShapeDtypeStruct((b, n_patch, n_out), f32),
        grid=(b // bs,),
        in_specs=in_specs,
        out_specs=pl.BlockSpec((bs, n_patch, n_out), lambda i: (i, 0, 0)),
        compiler_params=pltpu.CompilerParams(
            dimension_semantics=("parallel",)),
    )(x.astype(f32), *prepped)


# speedup vs baseline: 1.2005x; 1.0060x over previous
import functools

import jax
import jax.numpy as jnp
from jax import lax
from jax.experimental import pallas as pl
from jax.experimental.pallas import tpu as pltpu

_LN_EPS = 1e-5
_BS = 8


_INV_SQRT2 = 0.7071067811865476


def _gelu_pre(h):
    return h * (1.0 + lax.erf(h))


def _ln_stats(x):
    m = jnp.mean(x, axis=-1, keepdims=True)
    xc = x - m
    var = jnp.mean(xc * xc, axis=-1, keepdims=True)
    return m, lax.rsqrt(var + _LN_EPS)


def _mixer_body(depth, bs,
                x_ref, ln1g, ln1b, wt1, bt1, wt2, bt2,
                wc1p, bc1p, wc2t, bc2, whp, bhp, o_ref):
    bf16 = jnp.bfloat16
    f32 = jnp.float32
    for b in range(bs):
        x = x_ref[b]
        for d in range(depth):
            m, r = _ln_stats(x)
            y = ((x - m) * r * ln1g[d] + ln1b[d]).astype(bf16)
            h = jnp.dot(wt1[d], y, preferred_element_type=f32) + bt1[d]
            h = _gelu_pre(h).astype(bf16)
            x = x + (jnp.dot(wt2[d], h, preferred_element_type=f32)
                     + bt2[d])
            m, r = _ln_stats(x)
            z = ((x - m) * r).astype(bf16)
            h = jnp.dot(z, wc1p[d], preferred_element_type=f32) + bc1p[d]
            h = _gelu_pre(h).astype(bf16)
            x = x + (jnp.dot(h, wc2t[d], preferred_element_type=f32)
                     + bc2[d])
        m, r = _ln_stats(x)
        z = ((x - m) * r).astype(bf16)
        o_ref[b] = jnp.dot(z, whp[...], preferred_element_type=f32) + bhp[...]


def _rep_spec(shape):
    nd = len(shape)
    return pl.BlockSpec(shape, lambda i, _n=nd: (0,) * _n)


@jax.jit
def kernel(x, ln1_g, ln1_b, wt1, bt1, wt2, bt2, ln2_g, ln2_b,
           wc1, bc1, wc2, bc2, lnf_g, lnf_b, wh, bh):
    b, n_patch, dim = x.shape
    depth = wt1.shape[0]
    n_out = wh.shape[0]
    f32, bf16 = jnp.float32, jnp.bfloat16
    bs = _BS if b % _BS == 0 else 1

    s = _INV_SQRT2
    wc1t = jnp.transpose(wc1.astype(f32), (0, 2, 1))
    wc1p = (s * ln2_g.astype(f32)[:, :, None] * wc1t).astype(bf16)
    bc1p = (s * (bc1.astype(f32)
                 + jnp.einsum('dk,dkc->dc', ln2_b.astype(f32), wc1t)))[:, None, :]
    wc2t = (s * jnp.transpose(wc2.astype(f32), (0, 2, 1))).astype(bf16)
    wht = wh.astype(f32).T
    whp = (lnf_g.astype(f32)[:, None] * wht).astype(bf16)
    bhp = (bh.astype(f32) + lnf_b.astype(f32) @ wht)[None, :]

    prepped = [
        ln1_g.astype(f32).reshape(depth, 1, dim),
        ln1_b.astype(f32).reshape(depth, 1, dim),
        (s * wt1.astype(f32)).astype(bf16),
        (s * bt1.astype(f32))[:, :, None],
        (s * wt2.astype(f32)).astype(bf16),
        bt2.astype(f32)[:, :, None],
        wc1p, bc1p,
        wc2t,
        bc2.astype(f32)[:, None, :],
        whp, bhp,
    ]

    in_specs = [pl.BlockSpec((bs, n_patch, dim), lambda i: (i, 0, 0))]
    in_specs += [_rep_spec(a.shape) for a in prepped]

    return pl.pallas_call(
        functools.partial(_mixer_body, depth, bs),
        out_shape=jax.ShapeDtypeStruct((b, n_patch, n_out), f32),
        grid=(b // bs,),
        in_specs=in_specs,
        out_specs=pl.BlockSpec((bs, n_patch, n_out), lambda i: (i, 0, 0)),
        compiler_params=pltpu.CompilerParams(
            dimension_semantics=("parallel",)),
    )(x.astype(f32), *prepped)


# speedup vs baseline: 2.3485x; 1.9562x over previous
import functools

import jax
import jax.numpy as jnp
from jax import lax
from jax.experimental import pallas as pl
from jax.experimental.pallas import tpu as pltpu

_LN_EPS = 1e-5
_BS = 8


_INV_SQRT2 = 0.7071067811865476


def _gelu_pre(h):
    return h * (1.0 + lax.erf(h))


def _ln_stats(x):
    m = jnp.mean(x, axis=-1, keepdims=True)
    msq = jnp.mean(x * x, axis=-1, keepdims=True)
    var = msq - m * m
    return m, lax.rsqrt(var + _LN_EPS)


def _mixer_body(depth, bs,
                x_ref, ln1g, ln1b, wt1, bt1, wt2, bt2,
                wc1p, bc1p, wc2t, bc2, whp, bhp, o_ref):
    bf16 = jnp.bfloat16
    f32 = jnp.float32
    nc, dim = x_ref.shape[1], x_ref.shape[2]
    x = x_ref[...].reshape(bs * nc, dim)
    for d in range(depth):
        m, r = _ln_stats(x)
        y = ((x - m) * r * ln1g[d] + ln1b[d]).astype(bf16)
        y3 = y.reshape(bs, nc, dim)
        upd = []
        for b in range(bs):
            h = jnp.dot(wt1[d], y3[b], preferred_element_type=f32) + bt1[d]
            h = _gelu_pre(h).astype(bf16)
            upd.append(jnp.dot(wt2[d], h, preferred_element_type=f32)
                       + bt2[d])
        x = x + jnp.stack(upd).reshape(bs * nc, dim)
        m, r = _ln_stats(x)
        z = ((x - m) * r).astype(bf16)
        h = jnp.dot(z, wc1p[d], preferred_element_type=f32) + bc1p[d]
        h = _gelu_pre(h).astype(bf16)
        x = x + (jnp.dot(h, wc2t[d], preferred_element_type=f32)
                 + bc2[d])
    m, r = _ln_stats(x)
    z = ((x - m) * r).astype(bf16)
    out = jnp.dot(z, whp[...], preferred_element_type=f32) + bhp[...]
    o_ref[...] = out.reshape(bs, nc, out.shape[-1])


def _rep_spec(shape):
    nd = len(shape)
    return pl.BlockSpec(shape, lambda i, _n=nd: (0,) * _n)


@jax.jit
def kernel(x, ln1_g, ln1_b, wt1, bt1, wt2, bt2, ln2_g, ln2_b,
           wc1, bc1, wc2, bc2, lnf_g, lnf_b, wh, bh):
    b, n_patch, dim = x.shape
    depth = wt1.shape[0]
    n_out = wh.shape[0]
    f32, bf16 = jnp.float32, jnp.bfloat16
    bs = _BS if b % _BS == 0 else 1

    s = _INV_SQRT2
    wc1t = jnp.transpose(wc1.astype(f32), (0, 2, 1))
    wc1p = (s * ln2_g.astype(f32)[:, :, None] * wc1t).astype(bf16)
    bc1p = (s * (bc1.astype(f32)
                 + jnp.einsum('dk,dkc->dc', ln2_b.astype(f32), wc1t)))[:, None, :]
    wc2t = (s * jnp.transpose(wc2.astype(f32), (0, 2, 1))).astype(bf16)
    wht = wh.astype(f32).T
    whp = (lnf_g.astype(f32)[:, None] * wht).astype(bf16)
    bhp = (bh.astype(f32) + lnf_b.astype(f32) @ wht)[None, :]

    prepped = [
        ln1_g.astype(f32).reshape(depth, 1, dim),
        ln1_b.astype(f32).reshape(depth, 1, dim),
        (s * wt1.astype(f32)).astype(bf16),
        (s * bt1.astype(f32))[:, :, None],
        (s * wt2.astype(f32)).astype(bf16),
        bt2.astype(f32)[:, :, None],
        wc1p, bc1p,
        wc2t,
        bc2.astype(f32)[:, None, :],
        whp, bhp,
    ]

    in_specs = [pl.BlockSpec((bs, n_patch, dim), lambda i: (i, 0, 0))]
    in_specs += [_rep_spec(a.shape) for a in prepped]

    return pl.pallas_call(
        functools.partial(_mixer_body, depth, bs),
        out_shape=jax.ShapeDtypeStruct((b, n_patch, n_out), f32),
        grid=(b // bs,),
        in_specs=in_specs,
        out_specs=pl.BlockSpec((bs, n_patch, n_out), lambda i: (i, 0, 0)),
        compiler_params=pltpu.CompilerParams(
            dimension_semantics=("parallel",)),
    )(x.astype(f32), *prepped)


# speedup vs baseline: 2.4126x; 1.0273x over previous
import functools

import jax
import jax.numpy as jnp
from jax import lax
from jax.experimental import pallas as pl
from jax.experimental.pallas import tpu as pltpu

_LN_EPS = 1e-5
_BS = 8


_INV_SQRT2 = 0.7071067811865476


def _gelu_pre(h):
    one = jnp.asarray(1.0, h.dtype)
    return h * (one + lax.erf(h))


def _ln_stats(x):
    m = jnp.mean(x, axis=-1, keepdims=True)
    msq = jnp.mean(x * x, axis=-1, keepdims=True)
    var = msq - m * m
    return m, lax.rsqrt(var + _LN_EPS)


def _mixer_body(depth, bs,
                x_ref, ln1g, ln1b, wt1, bt1, wt2, bt2,
                wc1p, bc1p, wc2t, bc2, whp, bhp, o_ref):
    bf16 = jnp.bfloat16
    f32 = jnp.float32
    nc, dim = x_ref.shape[1], x_ref.shape[2]
    x = x_ref[...].reshape(bs * nc, dim)
    for d in range(depth):
        m, r = _ln_stats(x)
        y = ((x - m) * r * ln1g[d] + ln1b[d]).astype(bf16)
        y3 = y.reshape(bs, nc, dim)
        upd = []
        for b in range(bs):
            h = (jnp.dot(wt1[d], y3[b], preferred_element_type=f32)
                 ).astype(bf16) + bt1[d]
            h = _gelu_pre(h)
            upd.append(jnp.dot(wt2[d], h, preferred_element_type=f32)
                       + bt2[d])
        x = x + jnp.stack(upd).reshape(bs * nc, dim)
        m, r = _ln_stats(x)
        z = ((x - m) * r).astype(bf16)
        h = (jnp.dot(z, wc1p[d], preferred_element_type=f32)
             ).astype(bf16) + bc1p[d]
        h = _gelu_pre(h)
        x = x + (jnp.dot(h, wc2t[d], preferred_element_type=f32)
                 + bc2[d])
    m, r = _ln_stats(x)
    z = ((x - m) * r).astype(bf16)
    out = jnp.dot(z, whp[...], preferred_element_type=f32) + bhp[...]
    o_ref[...] = out.reshape(bs, nc, out.shape[-1])


def _rep_spec(shape):
    nd = len(shape)
    return pl.BlockSpec(shape, lambda i, _n=nd: (0,) * _n)


@jax.jit
def kernel(x, ln1_g, ln1_b, wt1, bt1, wt2, bt2, ln2_g, ln2_b,
           wc1, bc1, wc2, bc2, lnf_g, lnf_b, wh, bh):
    b, n_patch, dim = x.shape
    depth = wt1.shape[0]
    n_out = wh.shape[0]
    f32, bf16 = jnp.float32, jnp.bfloat16
    bs = _BS if b % _BS == 0 else 1

    s = _INV_SQRT2
    wc1t = jnp.transpose(wc1.astype(f32), (0, 2, 1))
    wc1p = (s * ln2_g.astype(f32)[:, :, None] * wc1t).astype(bf16)
    bc1p = (s * (bc1.astype(f32)
                 + jnp.einsum('dk,dkc->dc', ln2_b.astype(f32),
                              wc1t)))[:, None, :].astype(bf16)
    wc2t = (s * jnp.transpose(wc2.astype(f32), (0, 2, 1))).astype(bf16)
    wht = wh.astype(f32).T
    whp = (lnf_g.astype(f32)[:, None] * wht).astype(bf16)
    bhp = (bh.astype(f32) + lnf_b.astype(f32) @ wht)[None, :]

    prepped = [
        ln1_g.astype(f32).reshape(depth, 1, dim),
        ln1_b.astype(f32).reshape(depth, 1, dim),
        (s * wt1.astype(f32)).astype(bf16),
        (s * bt1.astype(f32))[:, :, None].astype(bf16),
        (s * wt2.astype(f32)).astype(bf16),
        bt2.astype(f32)[:, :, None],
        wc1p, bc1p,
        wc2t,
        bc2.astype(f32)[:, None, :],
        whp, bhp,
    ]

    in_specs = [pl.BlockSpec((bs, n_patch, dim), lambda i: (i, 0, 0))]
    in_specs += [_rep_spec(a.shape) for a in prepped]

    return pl.pallas_call(
        functools.partial(_mixer_body, depth, bs),
        out_shape=jax.ShapeDtypeStruct((b, n_patch, n_out), f32),
        grid=(b // bs,),
        in_specs=in_specs,
        out_specs=pl.BlockSpec((bs, n_patch, n_out), lambda i: (i, 0, 0)),
        compiler_params=pltpu.CompilerParams(
            dimension_semantics=("parallel",)),
    )(x.astype(f32), *prepped)


# speedup vs baseline: 2.4745x; 1.0257x over previous
import functools

import jax
import jax.numpy as jnp
from jax import lax
from jax.experimental import pallas as pl
from jax.experimental.pallas import tpu as pltpu

_LN_EPS = 1e-5
_BS = 16


_INV_SQRT2 = 0.7071067811865476


def _gelu_pre(h):
    one = jnp.asarray(1.0, h.dtype)
    return h * (one + lax.erf(h))


def _ln_stats(x):
    m = jnp.mean(x, axis=-1, keepdims=True)
    msq = jnp.mean(x * x, axis=-1, keepdims=True)
    var = msq - m * m
    return m, lax.rsqrt(var + _LN_EPS)


def _mixer_body(depth, bs,
                x_ref, ln1g, ln1b, wt1, bt1, wt2, bt2,
                wc1p, bc1p, wc2t, bc2, whp, bhp, o_ref):
    bf16 = jnp.bfloat16
    f32 = jnp.float32
    nc, dim = x_ref.shape[1], x_ref.shape[2]
    x = x_ref[...].reshape(bs * nc, dim)
    for d in range(depth):
        m, r = _ln_stats(x)
        y = ((x - m) * r * ln1g[d] + ln1b[d]).astype(bf16)
        y3 = y.reshape(bs, nc, dim)
        upd = []
        for b in range(bs):
            h = (jnp.dot(wt1[d], y3[b], preferred_element_type=f32)
                 ).astype(bf16) + bt1[d]
            h = _gelu_pre(h)
            upd.append(jnp.dot(wt2[d], h, preferred_element_type=f32)
                       + bt2[d])
        x = x + jnp.stack(upd).reshape(bs * nc, dim)
        m, r = _ln_stats(x)
        z = ((x - m) * r).astype(bf16)
        h = (jnp.dot(z, wc1p[d], preferred_element_type=f32)
             ).astype(bf16) + bc1p[d]
        h = _gelu_pre(h)
        x = x + (jnp.dot(h, wc2t[d], preferred_element_type=f32)
                 + bc2[d])
    m, r = _ln_stats(x)
    z = ((x - m) * r).astype(bf16)
    out = jnp.dot(z, whp[...], preferred_element_type=f32) + bhp[...]
    o_ref[...] = out.reshape(bs, nc, out.shape[-1])


def _rep_spec(shape):
    nd = len(shape)
    return pl.BlockSpec(shape, lambda i, _n=nd: (0,) * _n)


@jax.jit
def kernel(x, ln1_g, ln1_b, wt1, bt1, wt2, bt2, ln2_g, ln2_b,
           wc1, bc1, wc2, bc2, lnf_g, lnf_b, wh, bh):
    b, n_patch, dim = x.shape
    depth = wt1.shape[0]
    n_out = wh.shape[0]
    f32, bf16 = jnp.float32, jnp.bfloat16
    bs = _BS if b % _BS == 0 else 1

    s = _INV_SQRT2
    wc1t = jnp.transpose(wc1.astype(f32), (0, 2, 1))
    wc1p = (s * ln2_g.astype(f32)[:, :, None] * wc1t).astype(bf16)
    bc1p = (s * (bc1.astype(f32)
                 + jnp.einsum('dk,dkc->dc', ln2_b.astype(f32),
                              wc1t)))[:, None, :].astype(bf16)
    wc2t = (s * jnp.transpose(wc2.astype(f32), (0, 2, 1))).astype(bf16)
    wht = wh.astype(f32).T
    whp = (lnf_g.astype(f32)[:, None] * wht).astype(bf16)
    bhp = (bh.astype(f32) + lnf_b.astype(f32) @ wht)[None, :]

    prepped = [
        ln1_g.astype(f32).reshape(depth, 1, dim),
        ln1_b.astype(f32).reshape(depth, 1, dim),
        (s * wt1.astype(f32)).astype(bf16),
        (s * bt1.astype(f32))[:, :, None].astype(bf16),
        (s * wt2.astype(f32)).astype(bf16),
        bt2.astype(f32)[:, :, None],
        wc1p, bc1p,
        wc2t,
        bc2.astype(f32)[:, None, :],
        whp, bhp,
    ]

    in_specs = [pl.BlockSpec((bs, n_patch, dim), lambda i: (i, 0, 0))]
    in_specs += [_rep_spec(a.shape) for a in prepped]

    return pl.pallas_call(
        functools.partial(_mixer_body, depth, bs),
        out_shape=jax.ShapeDtypeStruct((b, n_patch, n_out), f32),
        grid=(b // bs,),
        in_specs=in_specs,
        out_specs=pl.BlockSpec((bs, n_patch, n_out), lambda i: (i, 0, 0)),
        compiler_params=pltpu.CompilerParams(
            dimension_semantics=("parallel",)),
    )(x.astype(f32), *prepped)
